# baseline (speedup 1.0000x reference)
import functools
import math

import jax
import jax.numpy as jnp
from jax import lax
from jax.experimental import pallas as pl
from jax.experimental.pallas import tpu as pltpu
from jax.experimental.pallas import tpu_sc as plsc

NTIPS = 256
HID = 256
LAT = 50
B = 4
Z = 32
NDIM = 509
NNODE = 510
ZB = 8
NPAD = 512
LOG2PI = math.log(2.0 * math.pi)


def _elu(x):
    return jnp.where(x > 0, x, jnp.exp(jnp.minimum(x, 0.0)) - 1.0)


def _mlp_kernel(h_ref, w1_ref, b1_ref, w2_ref, b2_ref, ms_ref):
    h = h_ref[0]
    x = _elu(jnp.dot(h, w1_ref[...], preferred_element_type=jnp.float32)
             + b1_ref[...])
    ms_ref[0] = _elu(jnp.dot(x, w2_ref[...], preferred_element_type=jnp.float32)
                     + b2_ref[...])


def _parent_gather(ms_flat, pi_glob):
    info = plsc.get_sparse_core_info()
    nw = info.num_cores * info.num_subcores
    rows_per_w = (B * NPAD) // nw
    mesh = plsc.VectorSubcoreMesh(core_axis_name="c", subcore_axis_name="s")

    @functools.partial(
        pl.kernel,
        out_type=jax.ShapeDtypeStruct((B * NPAD, HID), jnp.float32),
        mesh=mesh,
        scratch_types=[
            pltpu.VMEM((rows_per_w,), jnp.int32),
            pltpu.VMEM((rows_per_w, HID), jnp.float32),
            pltpu.SemaphoreType.DMA,
        ],
    )
    def gather_k(ms_hbm, idx_hbm, out_hbm, idx_v, rows_v, sem):
        wid = lax.axis_index("s") * info.num_cores + lax.axis_index("c")
        base = wid * rows_per_w
        pltpu.sync_copy(idx_hbm.at[pl.ds(base, rows_per_w)], idx_v)
        pltpu.async_copy(ms_hbm.at[idx_v], rows_v, sem).wait()
        pltpu.sync_copy(rows_v, out_hbm.at[pl.ds(base, rows_per_w)])

    return gather_k(ms_flat, pi_glob)


def _readout_kernel(ms_ref, pf_ref, wr1a_ref, br1_ref, sz_ref, wr1b_ref,
                    wr2_ref, br2_ref, sxr_ref, slb_ref, logq_ref,
                    base_s, slb_s):
    zb = pl.program_id(1)

    @pl.when(zb == 0)
    def _():
        mst = jnp.maximum(ms_ref[0, :NDIM, :], pf_ref[0, :NDIM, :])
        base_s[...] = lax.dot_general(
            wr1a_ref[...], mst, (((0,), (1,)), ((), ())),
            preferred_element_type=jnp.float32) + br1_ref[...]

    base = base_s[...]
    logqs = []
    for k in range(ZB):
        sz = sz_ref[0, k]
        zc = lax.dot_general(wr1b_ref[...], sz, (((0,), (1,)), ((), ())),
                             preferred_element_type=jnp.float32)
        r = _elu(base + zc)
        out = lax.dot_general(wr2_ref[...], r, (((0,), (0,)), ((), ())),
                              preferred_element_type=jnp.float32) + br2_ref[...]
        mean = out[0:1, :]
        ls = jnp.maximum(out[1:2, :], -3.0)

        if k == 0:
            @pl.when(zb == 0)
            def _():
                slb_s[...] = sxr_ref[0] * jnp.exp(ls) + mean - 2.0
                slb_ref[0] = slb_s[...]

        slb = slb_s[...]
        dev = (slb - mean + 2.0) * jnp.exp(-ls)
        logq = -0.5 * jnp.sum(LOG2PI + dev * dev) - jnp.sum(ls)
        logqs.append(jnp.full((1, 1, 1, 128), logq, jnp.float32))
    logq_ref[...] = jnp.concatenate(logqs, axis=1)


def kernel(node_features, parent_index, samp_z, samp_x_raw, W1, b1, W2, b2,
           Wr1, br1, Wr2, br2):
    f32 = jnp.float32
    sxr = samp_x_raw.reshape(B, 1, NDIM)
    b1r = b1.reshape(1, HID)
    b2r = b2.reshape(1, HID)
    wr1a = Wr1[:HID]
    wr1b = Wr1[HID:]
    br1c = br1.reshape(HID, 1)
    br2c = br2.reshape(2, 1)
    pi32 = parent_index.astype(jnp.int32)
    pi_glob = jnp.pad(
        pi32 + (jnp.arange(B, dtype=jnp.int32) * NNODE)[:, None],
        ((0, 0), (0, NPAD - NDIM)),
    ).reshape(B * NPAD)

    ms = pl.pallas_call(
        _mlp_kernel,
        grid=(B,),
        in_specs=[
            pl.BlockSpec((1, NNODE, NTIPS), lambda b: (b, 0, 0)),
            pl.BlockSpec((NTIPS, HID), lambda b: (0, 0)),
            pl.BlockSpec((1, HID), lambda b: (0, 0)),
            pl.BlockSpec((HID, HID), lambda b: (0, 0)),
            pl.BlockSpec((1, HID), lambda b: (0, 0)),
        ],
        out_specs=pl.BlockSpec((1, NNODE, HID), lambda b: (b, 0, 0)),
        out_shape=jax.ShapeDtypeStruct((B, NNODE, HID), f32),
    )(node_features, W1, b1r, W2, b2r)

    pf = _parent_gather(ms.reshape(B * NNODE, HID), pi_glob)
    pf = pf.reshape(B, NPAD, HID)

    slb_p, logq_p = pl.pallas_call(
        _readout_kernel,
        grid=(B, Z // ZB),
        in_specs=[
            pl.BlockSpec((1, NNODE, HID), lambda b, z: (b, 0, 0)),
            pl.BlockSpec((1, NPAD, HID), lambda b, z: (b, 0, 0)),
            pl.BlockSpec((HID, HID), lambda b, z: (0, 0)),
            pl.BlockSpec((HID, 1), lambda b, z: (0, 0)),
            pl.BlockSpec((1, ZB, NDIM, LAT), lambda b, z: (b, z, 0, 0)),
            pl.BlockSpec((LAT, HID), lambda b, z: (0, 0)),
            pl.BlockSpec((HID, 2), lambda b, z: (0, 0)),
            pl.BlockSpec((2, 1), lambda b, z: (0, 0)),
            pl.BlockSpec((1, 1, NDIM), lambda b, z: (b, 0, 0)),
        ],
        out_specs=[
            pl.BlockSpec((1, 1, NDIM), lambda b, z: (b, 0, 0)),
            pl.BlockSpec((1, ZB, 1, 128), lambda b, z: (b, z, 0, 0)),
        ],
        out_shape=[
            jax.ShapeDtypeStruct((B, 1, NDIM), f32),
            jax.ShapeDtypeStruct((B, Z, 1, 128), f32),
        ],
        scratch_shapes=[pltpu.VMEM((HID, NDIM), f32), pltpu.VMEM((1, NDIM), f32)],
    )(ms, pf, wr1a, br1c, samp_z, wr1b, Wr2, br2c, sxr)

    samp_log_branch = slb_p[:, 0, :]
    logq_branch_batch = logq_p[:, :, 0, 0]
    return (samp_log_branch, logq_branch_batch)


# speedup vs baseline: 1.0312x; 1.0312x over previous
import functools
import math

import jax
import jax.numpy as jnp
from jax import lax
from jax.experimental import pallas as pl
from jax.experimental.pallas import tpu as pltpu
from jax.experimental.pallas import tpu_sc as plsc

NTIPS = 256
HID = 256
LAT = 50
B = 4
Z = 32
NDIM = 509
NNODE = 510
ZB = 8
NPAD = 512
LOG2PI = math.log(2.0 * math.pi)


def _elu(x):
    return jnp.where(x > 0, x, jnp.exp(jnp.minimum(x, 0.0)) - 1.0)


def _parent_gather(h_flat, pi_glob):
    info = plsc.get_sparse_core_info()
    nw = info.num_cores * info.num_subcores
    rows_per_w = (B * NPAD) // nw
    mesh = plsc.VectorSubcoreMesh(core_axis_name="c", subcore_axis_name="s")

    @functools.partial(
        pl.kernel,
        out_type=jax.ShapeDtypeStruct((B * NPAD, NTIPS), jnp.float32),
        mesh=mesh,
        scratch_types=[
            pltpu.VMEM((rows_per_w,), jnp.int32),
            pltpu.VMEM((rows_per_w, NTIPS), jnp.float32),
            pltpu.SemaphoreType.DMA,
        ],
    )
    def gather_k(h_hbm, idx_hbm, out_hbm, idx_v, rows_v, sem):
        wid = lax.axis_index("s") * info.num_cores + lax.axis_index("c")
        base = wid * rows_per_w
        pltpu.sync_copy(idx_hbm.at[pl.ds(base, rows_per_w)], idx_v)
        pltpu.async_copy(h_hbm.at[idx_v], rows_v, sem).wait()
        pltpu.sync_copy(rows_v, out_hbm.at[pl.ds(base, rows_per_w)])

    return gather_k(h_flat, pi_glob)


def _front_kernel(h_ref, hp_ref, w1_ref, b1_ref, w2_ref, b2_ref,
                  wr1a_ref, br1_ref, base_ref):
    w1 = w1_ref[...]
    w2 = w2_ref[...]
    b1 = b1_ref[...]
    b2 = b2_ref[...]

    def mlp(x):
        y = _elu(jnp.dot(x, w1, preferred_element_type=jnp.float32) + b1)
        return _elu(jnp.dot(y, w2, preferred_element_type=jnp.float32) + b2)

    ms = mlp(h_ref[0, :NDIM, :])
    msp = mlp(hp_ref[0, :NDIM, :])
    mst = jnp.maximum(ms, msp)
    base_ref[0] = lax.dot_general(
        wr1a_ref[...], mst, (((0,), (1,)), ((), ())),
        preferred_element_type=jnp.float32) + br1_ref[...]


def _readout_kernel(base_ref, sz_ref, wr1b_ref, wr2_ref, br2_ref, sxr_ref,
                    slb_ref, logq_ref, slb_s):
    zb = pl.program_id(1)
    base = base_ref[0]
    logqs = []
    for k in range(ZB):
        sz = sz_ref[0, k]
        zc = lax.dot_general(wr1b_ref[...], sz, (((0,), (1,)), ((), ())),
                             preferred_element_type=jnp.float32)
        r = _elu(base + zc)
        out = lax.dot_general(wr2_ref[...], r, (((0,), (0,)), ((), ())),
                              preferred_element_type=jnp.float32) + br2_ref[...]
        mean = out[0:1, :]
        ls = jnp.maximum(out[1:2, :], -3.0)

        if k == 0:
            @pl.when(zb == 0)
            def _():
                slb_s[...] = sxr_ref[0] * jnp.exp(ls) + mean - 2.0
                slb_ref[0] = slb_s[...]

        slb = slb_s[...]
        dev = (slb - mean + 2.0) * jnp.exp(-ls)
        logq = -0.5 * jnp.sum(LOG2PI + dev * dev) - jnp.sum(ls)
        logqs.append(jnp.full((1, 1, 1, 128), logq, jnp.float32))
    logq_ref[...] = jnp.concatenate(logqs, axis=1)


def kernel(node_features, parent_index, samp_z, samp_x_raw, W1, b1, W2, b2,
           Wr1, br1, Wr2, br2):
    f32 = jnp.float32
    sxr = samp_x_raw.reshape(B, 1, NDIM)
    b1r = b1.reshape(1, HID)
    b2r = b2.reshape(1, HID)
    wr1a = Wr1[:HID]
    wr1b = Wr1[HID:]
    br1c = br1.reshape(HID, 1)
    br2c = br2.reshape(2, 1)
    pi32 = parent_index.astype(jnp.int32)
    pi_glob = jnp.pad(
        pi32 + (jnp.arange(B, dtype=jnp.int32) * NNODE)[:, None],
        ((0, 0), (0, NPAD - NDIM)),
    ).reshape(B * NPAD)

    hp = _parent_gather(node_features.reshape(B * NNODE, NTIPS), pi_glob)
    hp = hp.reshape(B, NPAD, NTIPS)

    base = pl.pallas_call(
        _front_kernel,
        grid=(B,),
        in_specs=[
            pl.BlockSpec((1, NNODE, NTIPS), lambda b: (b, 0, 0)),
            pl.BlockSpec((1, NPAD, NTIPS), lambda b: (b, 0, 0)),
            pl.BlockSpec((NTIPS, HID), lambda b: (0, 0)),
            pl.BlockSpec((1, HID), lambda b: (0, 0)),
            pl.BlockSpec((HID, HID), lambda b: (0, 0)),
            pl.BlockSpec((1, HID), lambda b: (0, 0)),
            pl.BlockSpec((HID, HID), lambda b: (0, 0)),
            pl.BlockSpec((HID, 1), lambda b: (0, 0)),
        ],
        out_specs=pl.BlockSpec((1, HID, NDIM), lambda b: (b, 0, 0)),
        out_shape=jax.ShapeDtypeStruct((B, HID, NDIM), f32),
    )(node_features, hp, W1, b1r, W2, b2r, wr1a, br1c)

    slb_p, logq_p = pl.pallas_call(
        _readout_kernel,
        grid=(B, Z // ZB),
        in_specs=[
            pl.BlockSpec((1, HID, NDIM), lambda b, z: (b, 0, 0)),
            pl.BlockSpec((1, ZB, NDIM, LAT), lambda b, z: (b, z, 0, 0)),
            pl.BlockSpec((LAT, HID), lambda b, z: (0, 0)),
            pl.BlockSpec((HID, 2), lambda b, z: (0, 0)),
            pl.BlockSpec((2, 1), lambda b, z: (0, 0)),
            pl.BlockSpec((1, 1, NDIM), lambda b, z: (b, 0, 0)),
        ],
        out_specs=[
            pl.BlockSpec((1, 1, NDIM), lambda b, z: (b, 0, 0)),
            pl.BlockSpec((1, ZB, 1, 128), lambda b, z: (b, z, 0, 0)),
        ],
        out_shape=[
            jax.ShapeDtypeStruct((B, 1, NDIM), f32),
            jax.ShapeDtypeStruct((B, Z, 1, 128), f32),
        ],
        scratch_shapes=[pltpu.VMEM((1, NDIM), f32)],
    )(base, samp_z, wr1b, Wr2, br2c, sxr)

    samp_log_branch = slb_p[:, 0, :]
    logq_branch_batch = logq_p[:, :, 0, 0]
    return (samp_log_branch, logq_branch_batch)


# speedup vs baseline: 1.0525x; 1.0207x over previous
import functools
import math

import jax
import jax.numpy as jnp
from jax import lax
from jax.experimental import pallas as pl
from jax.experimental.pallas import tpu as pltpu
from jax.experimental.pallas import tpu_sc as plsc

NTIPS = 256
HID = 256
LAT = 50
B = 4
Z = 32
NDIM = 509
NNODE = 510
ZB = 8
NPAD = 512
LOG2PI = math.log(2.0 * math.pi)


def _elu(x):
    return jnp.where(x > 0, x, jnp.exp(jnp.minimum(x, 0.0)) - 1.0)


def _parent_gather(h_flat, pi_glob):
    info = plsc.get_sparse_core_info()
    nw = info.num_cores * info.num_subcores
    rows_per_w = (B * NPAD) // nw
    mesh = plsc.VectorSubcoreMesh(core_axis_name="c", subcore_axis_name="s")

    @functools.partial(
        pl.kernel,
        out_type=jax.ShapeDtypeStruct((B * NPAD, NTIPS), jnp.float32),
        mesh=mesh,
        scratch_types=[
            pltpu.VMEM((rows_per_w,), jnp.int32),
            pltpu.VMEM((rows_per_w, NTIPS), jnp.float32),
            pltpu.SemaphoreType.DMA,
        ],
    )
    def gather_k(h_hbm, idx_hbm, out_hbm, idx_v, rows_v, sem):
        wid = lax.axis_index("s") * info.num_cores + lax.axis_index("c")
        base = wid * rows_per_w
        pltpu.sync_copy(idx_hbm.at[pl.ds(base, rows_per_w)], idx_v)
        pltpu.async_copy(h_hbm.at[idx_v], rows_v, sem).wait()
        pltpu.sync_copy(rows_v, out_hbm.at[pl.ds(base, rows_per_w)])

    return gather_k(h_flat, pi_glob)


def _readout_kernel(h_ref, hp_ref, w1_ref, b1_ref, w2_ref, b2_ref,
                    wr1a_ref, br1_ref, sz_ref, wr1b_ref, wr2_ref, br2_ref,
                    sxr_ref, slb_ref, logq_ref, base_s, slb_s):
    zb = pl.program_id(1)

    @pl.when(zb == 0)
    def _():
        w1 = w1_ref[...]
        w2 = w2_ref[...]
        b1 = b1_ref[...]
        b2 = b2_ref[...]

        def mlp(x):
            y = _elu(jnp.dot(x, w1, preferred_element_type=jnp.float32) + b1)
            return _elu(jnp.dot(y, w2, preferred_element_type=jnp.float32) + b2)

        ms = mlp(h_ref[0, :NDIM, :])
        msp = mlp(hp_ref[0, :NDIM, :])
        mst = jnp.maximum(ms, msp)
        base_s[...] = lax.dot_general(
            wr1a_ref[...], mst, (((0,), (1,)), ((), ())),
            preferred_element_type=jnp.float32) + br1_ref[...]

    base = base_s[...]
    logqs = []
    for k in range(ZB):
        sz = sz_ref[0, k]
        zc = lax.dot_general(wr1b_ref[...], sz, (((0,), (1,)), ((), ())),
                             preferred_element_type=jnp.float32)
        r = _elu(base + zc)
        out = lax.dot_general(wr2_ref[...], r, (((0,), (0,)), ((), ())),
                              preferred_element_type=jnp.float32) + br2_ref[...]
        mean = out[0:1, :]
        ls = jnp.maximum(out[1:2, :], -3.0)

        if k == 0:
            @pl.when(zb == 0)
            def _():
                slb_s[...] = sxr_ref[0] * jnp.exp(ls) + mean - 2.0
                slb_ref[0] = slb_s[...]

        slb = slb_s[...]
        dev = (slb - mean + 2.0) * jnp.exp(-ls)
        logq = -0.5 * jnp.sum(LOG2PI + dev * dev) - jnp.sum(ls)
        logqs.append(jnp.full((1, 1, 1, 128), logq, jnp.float32))
    logq_ref[...] = jnp.concatenate(logqs, axis=1)


def kernel(node_features, parent_index, samp_z, samp_x_raw, W1, b1, W2, b2,
           Wr1, br1, Wr2, br2):
    f32 = jnp.float32
    sxr = samp_x_raw.reshape(B, 1, NDIM)
    b1r = b1.reshape(1, HID)
    b2r = b2.reshape(1, HID)
    wr1a = Wr1[:HID]
    wr1b = Wr1[HID:]
    br1c = br1.reshape(HID, 1)
    br2c = br2.reshape(2, 1)
    pi32 = parent_index.astype(jnp.int32)
    pi_glob = jnp.pad(
        pi32 + (jnp.arange(B, dtype=jnp.int32) * NNODE)[:, None],
        ((0, 0), (0, NPAD - NDIM)),
    ).reshape(B * NPAD)

    hp = _parent_gather(node_features.reshape(B * NNODE, NTIPS), pi_glob)
    hp = hp.reshape(B, NPAD, NTIPS)

    slb_p, logq_p = pl.pallas_call(
        _readout_kernel,
        grid=(B, Z // ZB),
        in_specs=[
            pl.BlockSpec((1, NNODE, NTIPS), lambda b, z: (b, 0, 0)),
            pl.BlockSpec((1, NPAD, NTIPS), lambda b, z: (b, 0, 0)),
            pl.BlockSpec((NTIPS, HID), lambda b, z: (0, 0)),
            pl.BlockSpec((1, HID), lambda b, z: (0, 0)),
            pl.BlockSpec((HID, HID), lambda b, z: (0, 0)),
            pl.BlockSpec((1, HID), lambda b, z: (0, 0)),
            pl.BlockSpec((HID, HID), lambda b, z: (0, 0)),
            pl.BlockSpec((HID, 1), lambda b, z: (0, 0)),
            pl.BlockSpec((1, ZB, NDIM, LAT), lambda b, z: (b, z, 0, 0)),
            pl.BlockSpec((LAT, HID), lambda b, z: (0, 0)),
            pl.BlockSpec((HID, 2), lambda b, z: (0, 0)),
            pl.BlockSpec((2, 1), lambda b, z: (0, 0)),
            pl.BlockSpec((1, 1, NDIM), lambda b, z: (b, 0, 0)),
        ],
        out_specs=[
            pl.BlockSpec((1, 1, NDIM), lambda b, z: (b, 0, 0)),
            pl.BlockSpec((1, ZB, 1, 128), lambda b, z: (b, z, 0, 0)),
        ],
        out_shape=[
            jax.ShapeDtypeStruct((B, 1, NDIM), f32),
            jax.ShapeDtypeStruct((B, Z, 1, 128), f32),
        ],
        scratch_shapes=[pltpu.VMEM((HID, NDIM), f32), pltpu.VMEM((1, NDIM), f32)],
    )(node_features, hp, W1, b1r, W2, b2r, wr1a, br1c, samp_z, wr1b, Wr2,
      br2c, sxr)

    samp_log_branch = slb_p[:, 0, :]
    logq_branch_batch = logq_p[:, :, 0, 0]
    return (samp_log_branch, logq_branch_batch)


# speedup vs baseline: 1.1048x; 1.0497x over previous
import functools
import math

import jax
import jax.numpy as jnp
from jax import lax
from jax.experimental import pallas as pl
from jax.experimental.pallas import tpu as pltpu
from jax.experimental.pallas import tpu_sc as plsc

NTIPS = 256
HID = 256
LAT = 50
B = 4
Z = 32
NDIM = 509
NNODE = 510
ZB = 16
NPAD = 512
LOG2PI = math.log(2.0 * math.pi)


def _elu(x):
    return jnp.where(x > 0, x, jnp.exp(x) - 1.0)


def _parent_gather(h_flat, pi_glob):
    info = plsc.get_sparse_core_info()
    nw = info.num_cores * info.num_subcores
    rows_per_w = (B * NPAD) // nw
    mesh = plsc.VectorSubcoreMesh(core_axis_name="c", subcore_axis_name="s")

    @functools.partial(
        pl.kernel,
        out_type=jax.ShapeDtypeStruct((B * NPAD, NTIPS), jnp.float32),
        mesh=mesh,
        scratch_types=[
            pltpu.VMEM((rows_per_w,), jnp.int32),
            pltpu.VMEM((rows_per_w, NTIPS), jnp.float32),
            pltpu.SemaphoreType.DMA,
        ],
    )
    def gather_k(h_hbm, idx_hbm, out_hbm, idx_v, rows_v, sem):
        wid = lax.axis_index("s") * info.num_cores + lax.axis_index("c")
        base = wid * rows_per_w
        pltpu.sync_copy(idx_hbm.at[pl.ds(base, rows_per_w)], idx_v)
        pltpu.async_copy(h_hbm.at[idx_v], rows_v, sem).wait()
        pltpu.sync_copy(rows_v, out_hbm.at[pl.ds(base, rows_per_w)])

    return gather_k(h_flat, pi_glob)


def _readout_kernel(h_ref, hp_ref, w1_ref, b1_ref, w2_ref, b2_ref,
                    wr1a_ref, br1_ref, sz_ref, wr1b_ref, wr2_ref, br2_ref,
                    sxr_ref, slb_ref, logq_ref, base_s, slb_s):
    zb = pl.program_id(1)

    @pl.when(zb == 0)
    def _():
        w1 = w1_ref[...]
        w2 = w2_ref[...]
        b1 = b1_ref[...]
        b2 = b2_ref[...]

        def mlp(x):
            y = _elu(jnp.dot(x, w1, preferred_element_type=jnp.float32) + b1)
            return _elu(jnp.dot(y, w2, preferred_element_type=jnp.float32) + b2)

        ms = mlp(h_ref[0, :NDIM, :])
        msp = mlp(hp_ref[0, :NDIM, :])
        mst = jnp.maximum(ms, msp)
        base_s[...] = lax.dot_general(
            wr1a_ref[...], mst, (((0,), (1,)), ((), ())),
            preferred_element_type=jnp.float32) + br1_ref[...]

    base = base_s[...]
    logqs = []
    for k in range(ZB):
        sz = sz_ref[0, k]
        zc = lax.dot_general(wr1b_ref[...], sz, (((0,), (1,)), ((), ())),
                             preferred_element_type=jnp.float32)
        r = _elu(base + zc)
        out = lax.dot_general(wr2_ref[...], r, (((0,), (0,)), ((), ())),
                              preferred_element_type=jnp.float32) + br2_ref[...]
        mean = out[0:1, :]
        ls = jnp.maximum(out[1:2, :], -3.0)

        if k == 0:
            @pl.when(zb == 0)
            def _():
                slb_s[...] = sxr_ref[0] * jnp.exp(ls) + mean - 2.0
                slb_ref[0] = slb_s[...]

        slb = slb_s[...]
        dev = (slb - mean + 2.0) * jnp.exp(-ls)
        logq = -0.5 * jnp.sum(LOG2PI + dev * dev) - jnp.sum(ls)
        logqs.append(jnp.full((1, 1, 1, 128), logq, jnp.float32))
    logq_ref[...] = jnp.concatenate(logqs, axis=1)


def kernel(node_features, parent_index, samp_z, samp_x_raw, W1, b1, W2, b2,
           Wr1, br1, Wr2, br2):
    f32 = jnp.float32
    sxr = samp_x_raw.reshape(B, 1, NDIM)
    b1r = b1.reshape(1, HID)
    b2r = b2.reshape(1, HID)
    wr1a = Wr1[:HID]
    wr1b = Wr1[HID:]
    br1c = br1.reshape(HID, 1)
    br2c = br2.reshape(2, 1)
    pi32 = parent_index.astype(jnp.int32)
    pi_glob = jnp.pad(
        pi32 + (jnp.arange(B, dtype=jnp.int32) * NNODE)[:, None],
        ((0, 0), (0, NPAD - NDIM)),
    ).reshape(B * NPAD)

    hp = _parent_gather(node_features.reshape(B * NNODE, NTIPS), pi_glob)
    hp = hp.reshape(B, NPAD, NTIPS)

    slb_p, logq_p = pl.pallas_call(
        _readout_kernel,
        grid=(B, Z // ZB),
        in_specs=[
            pl.BlockSpec((1, NNODE, NTIPS), lambda b, z: (b, 0, 0)),
            pl.BlockSpec((1, NPAD, NTIPS), lambda b, z: (b, 0, 0)),
            pl.BlockSpec((NTIPS, HID), lambda b, z: (0, 0)),
            pl.BlockSpec((1, HID), lambda b, z: (0, 0)),
            pl.BlockSpec((HID, HID), lambda b, z: (0, 0)),
            pl.BlockSpec((1, HID), lambda b, z: (0, 0)),
            pl.BlockSpec((HID, HID), lambda b, z: (0, 0)),
            pl.BlockSpec((HID, 1), lambda b, z: (0, 0)),
            pl.BlockSpec((1, ZB, NDIM, LAT), lambda b, z: (b, z, 0, 0)),
            pl.BlockSpec((LAT, HID), lambda b, z: (0, 0)),
            pl.BlockSpec((HID, 2), lambda b, z: (0, 0)),
            pl.BlockSpec((2, 1), lambda b, z: (0, 0)),
            pl.BlockSpec((1, 1, NDIM), lambda b, z: (b, 0, 0)),
        ],
        out_specs=[
            pl.BlockSpec((1, 1, NDIM), lambda b, z: (b, 0, 0)),
            pl.BlockSpec((1, ZB, 1, 128), lambda b, z: (b, z, 0, 0)),
        ],
        out_shape=[
            jax.ShapeDtypeStruct((B, 1, NDIM), f32),
            jax.ShapeDtypeStruct((B, Z, 1, 128), f32),
        ],
        scratch_shapes=[pltpu.VMEM((HID, NDIM), f32), pltpu.VMEM((1, NDIM), f32)],
    )(node_features, hp, W1, b1r, W2, b2r, wr1a, br1c, samp_z, wr1b, Wr2,
      br2c, sxr)

    samp_log_branch = slb_p[:, 0, :]
    logq_branch_batch = logq_p[:, :, 0, 0]
    return (samp_log_branch, logq_branch_batch)


# speedup vs baseline: 1.1882x; 1.0754x over previous
import functools
import math

import jax
import jax.numpy as jnp
from jax import lax
from jax.experimental import pallas as pl
from jax.experimental.pallas import tpu as pltpu
from jax.experimental.pallas import tpu_sc as plsc

NTIPS = 256
HID = 256
LAT = 50
B = 4
Z = 32
NDIM = 509
NNODE = 510
ZB = 32
NPAD = 512
LOG2PI = math.log(2.0 * math.pi)


def _elu(x):
    return jnp.where(x > 0, x, jnp.exp(x) - 1.0)


def _parent_gather(h_flat, pi_glob):
    info = plsc.get_sparse_core_info()
    nw = info.num_cores * info.num_subcores
    rows_per_w = (B * NPAD) // nw
    mesh = plsc.VectorSubcoreMesh(core_axis_name="c", subcore_axis_name="s")

    @functools.partial(
        pl.kernel,
        out_type=jax.ShapeDtypeStruct((B * NPAD, NTIPS), jnp.float32),
        mesh=mesh,
        scratch_types=[
            pltpu.VMEM((rows_per_w,), jnp.int32),
            pltpu.VMEM((rows_per_w, NTIPS), jnp.float32),
            pltpu.SemaphoreType.DMA,
        ],
    )
    def gather_k(h_hbm, idx_hbm, out_hbm, idx_v, rows_v, sem):
        wid = lax.axis_index("s") * info.num_cores + lax.axis_index("c")
        base = wid * rows_per_w
        pltpu.sync_copy(idx_hbm.at[pl.ds(base, rows_per_w)], idx_v)
        pltpu.async_copy(h_hbm.at[idx_v], rows_v, sem).wait()
        pltpu.sync_copy(rows_v, out_hbm.at[pl.ds(base, rows_per_w)])

    return gather_k(h_flat, pi_glob)


def _readout_kernel(h_ref, hp_ref, w1_ref, b1_ref, w2_ref, b2_ref,
                    wr1a_ref, br1_ref, sz_ref, wr1b_ref, wr2_ref, br2_ref,
                    sxr_ref, slb_ref, logq_ref, base_s, slb_s):
    zb = pl.program_id(1)

    @pl.when(zb == 0)
    def _():
        w1 = w1_ref[...]
        w2 = w2_ref[...]
        b1 = b1_ref[...]
        b2 = b2_ref[...]

        def mlp(x):
            y = _elu(jnp.dot(x, w1, preferred_element_type=jnp.float32) + b1)
            return _elu(jnp.dot(y, w2, preferred_element_type=jnp.float32) + b2)

        ms = mlp(h_ref[0, :NDIM, :])
        msp = mlp(hp_ref[0, :NDIM, :])
        mst = jnp.maximum(ms, msp)
        base_s[...] = lax.dot_general(
            wr1a_ref[...], mst, (((0,), (1,)), ((), ())),
            preferred_element_type=jnp.float32) + br1_ref[...]

    base = base_s[...]
    logqs = []
    for k in range(ZB):
        sz = sz_ref[0, k]
        zc = lax.dot_general(wr1b_ref[...], sz, (((0,), (1,)), ((), ())),
                             preferred_element_type=jnp.float32)
        r = _elu(base + zc)
        out = lax.dot_general(wr2_ref[...], r, (((0,), (0,)), ((), ())),
                              preferred_element_type=jnp.float32) + br2_ref[...]
        mean = out[0:1, :]
        ls = jnp.maximum(out[1:2, :], -3.0)

        if k == 0:
            @pl.when(zb == 0)
            def _():
                slb_s[...] = sxr_ref[0] * jnp.exp(ls) + mean - 2.0
                slb_ref[0] = slb_s[...]

        slb = slb_s[...]
        dev = (slb - mean + 2.0) * jnp.exp(-ls)
        logq = -0.5 * jnp.sum(LOG2PI + dev * dev) - jnp.sum(ls)
        logqs.append(jnp.full((1, 1, 1, 128), logq, jnp.float32))
    logq_ref[...] = jnp.concatenate(logqs, axis=1)


def kernel(node_features, parent_index, samp_z, samp_x_raw, W1, b1, W2, b2,
           Wr1, br1, Wr2, br2):
    f32 = jnp.float32
    sxr = samp_x_raw.reshape(B, 1, NDIM)
    b1r = b1.reshape(1, HID)
    b2r = b2.reshape(1, HID)
    wr1a = Wr1[:HID]
    wr1b = Wr1[HID:]
    br1c = br1.reshape(HID, 1)
    br2c = br2.reshape(2, 1)
    pi32 = parent_index.astype(jnp.int32)
    pi_glob = jnp.pad(
        pi32 + (jnp.arange(B, dtype=jnp.int32) * NNODE)[:, None],
        ((0, 0), (0, NPAD - NDIM)),
    ).reshape(B * NPAD)

    hp = _parent_gather(node_features.reshape(B * NNODE, NTIPS), pi_glob)
    hp = hp.reshape(B, NPAD, NTIPS)

    slb_p, logq_p = pl.pallas_call(
        _readout_kernel,
        grid=(B, Z // ZB),
        in_specs=[
            pl.BlockSpec((1, NNODE, NTIPS), lambda b, z: (b, 0, 0)),
            pl.BlockSpec((1, NPAD, NTIPS), lambda b, z: (b, 0, 0)),
            pl.BlockSpec((NTIPS, HID), lambda b, z: (0, 0)),
            pl.BlockSpec((1, HID), lambda b, z: (0, 0)),
            pl.BlockSpec((HID, HID), lambda b, z: (0, 0)),
            pl.BlockSpec((1, HID), lambda b, z: (0, 0)),
            pl.BlockSpec((HID, HID), lambda b, z: (0, 0)),
            pl.BlockSpec((HID, 1), lambda b, z: (0, 0)),
            pl.BlockSpec((1, ZB, NDIM, LAT), lambda b, z: (b, z, 0, 0)),
            pl.BlockSpec((LAT, HID), lambda b, z: (0, 0)),
            pl.BlockSpec((HID, 2), lambda b, z: (0, 0)),
            pl.BlockSpec((2, 1), lambda b, z: (0, 0)),
            pl.BlockSpec((1, 1, NDIM), lambda b, z: (b, 0, 0)),
        ],
        out_specs=[
            pl.BlockSpec((1, 1, NDIM), lambda b, z: (b, 0, 0)),
            pl.BlockSpec((1, ZB, 1, 128), lambda b, z: (b, z, 0, 0)),
        ],
        out_shape=[
            jax.ShapeDtypeStruct((B, 1, NDIM), f32),
            jax.ShapeDtypeStruct((B, Z, 1, 128), f32),
        ],
        scratch_shapes=[pltpu.VMEM((HID, NDIM), f32), pltpu.VMEM((1, NDIM), f32)],
    )(node_features, hp, W1, b1r, W2, b2r, wr1a, br1c, samp_z, wr1b, Wr2,
      br2c, sxr)

    samp_log_branch = slb_p[:, 0, :]
    logq_branch_batch = logq_p[:, :, 0, 0]
    return (samp_log_branch, logq_branch_batch)
